# baseline (speedup 1.0000x reference)
import functools

import jax
import jax.numpy as jnp
from jax import lax
from jax.experimental import pallas as pl
from jax.experimental.pallas import tpu as pltpu
from jax.experimental.pallas import tpu_sc as plsc

_NC = 2
_NS = 16
_NW = _NC * _NS
_L = 16

_C = 512


def kernel(feat, batch_ids, glob_feat):
    n = feat.shape[0]
    b, d = glob_feat.shape
    idx = batch_ids.astype(jnp.int32)
    glob_flat = glob_feat.reshape(-1)

    num_chunks = -(-n // _C)
    last_base = n - _C
    trips = -(-num_chunks // _NW)
    span = trips * _C
    idx_base_max = n - span
    ncg = d // _L

    mesh = plsc.VectorSubcoreMesh(
        core_axis_name="c", subcore_axis_name="s",
        num_cores=_NC, num_subcores=_NS,
    )

    @functools.partial(
        pl.kernel,
        out_type=jax.ShapeDtypeStruct((n * d,), jnp.float32),
        mesh=mesh,
        scratch_types=[
            pltpu.VMEM((span + _L,), jnp.int32),
            pltpu.VMEM((b * d,), jnp.float32),
            pltpu.VMEM((_C * d,), jnp.float32),
            pltpu.SemaphoreType.DMA,
        ],
    )
    def bcast(idx_hbm, glob_hbm, out_hbm, idx_v, glob_v, rows_v, sem_w):
        wid = lax.axis_index("s") * _NC + lax.axis_index("c")
        c0 = (wid * num_chunks) // _NW
        idx_base = jnp.minimum(c0 * _C, idx_base_max)

        cp_g = pltpu.async_copy(glob_hbm, glob_v, sem_w)
        cp_i = pltpu.async_copy(idx_hbm.at[pl.ds(idx_base, span)],
                                idx_v.at[pl.ds(0, span)], sem_w)
        cp_g.wait()
        cp_i.wait()

        def fill_uniform(bid):
            row = [glob_v[pl.ds(bid * d + k * _L, _L)] for k in range(ncg)]

            def store_row(r, carry):
                for k in range(ncg):
                    rows_v[pl.ds(r * d + k * _L, _L)] = row[k]
                return carry
            lax.fori_loop(0, _C, store_row, 0)

        def fill_mixed(voff):
            def store_row(r, carry):
                bid = idx_v[pl.ds(voff + r, _L)][0]
                for k in range(ncg):
                    rows_v[pl.ds(r * d + k * _L, _L)] = (
                        glob_v[pl.ds(bid * d + k * _L, _L)])
                return carry
            lax.fori_loop(0, _C, store_row, 0)

        def drain(k):
            def wait_one(_, carry):
                pltpu.make_async_copy(
                    rows_v, out_hbm.at[pl.ds(0, _C * d)], sem_w).wait()
                return carry
            lax.fori_loop(0, k, wait_one, 0)

        def body(j, carry):
            buf_id, n_out = carry
            cc = jnp.minimum(c0 + j, num_chunks - 1)
            base = jnp.minimum(cc * _C, last_base)
            voff = base - idx_base

            lo = idx_v[pl.ds(voff, _L)][0]
            hi = idx_v[pl.ds(voff + _C - _L, _L)][_L - 1]
            uniform = lo == hi
            stale = jnp.logical_not(jnp.logical_and(uniform, lo == buf_id))

            @pl.when(stale)
            def _():
                drain(n_out)

            @pl.when(jnp.logical_and(stale, uniform))
            def _():
                fill_uniform(lo)

            @pl.when(jnp.logical_not(uniform))
            def _():
                fill_mixed(voff)

            pltpu.async_copy(rows_v, out_hbm.at[pl.ds(base * d, _C * d)],
                             sem_w)
            new_id = jnp.where(uniform, lo, jnp.int32(-2))
            new_out = jnp.where(stale, jnp.int32(1), n_out + 1)
            return new_id, new_out

        _, n_left = lax.fori_loop(
            0, trips, body, (jnp.int32(-2), jnp.int32(0)))
        drain(n_left)

    out = bcast(idx, glob_flat)
    return out.reshape(n, d)


# speedup vs baseline: 1.4900x; 1.4900x over previous
import functools

import jax
import jax.numpy as jnp
from jax import lax
from jax.experimental import pallas as pl
from jax.experimental.pallas import tpu as pltpu
from jax.experimental.pallas import tpu_sc as plsc

_NC = 2
_NS = 16
_NW = _NC * _NS
_L = 16

_C = 128


def kernel(feat, batch_ids, glob_feat):
    n = feat.shape[0]
    b, d = glob_feat.shape
    idx = batch_ids.astype(jnp.int32)
    glob_flat = glob_feat.reshape(-1)

    num_chunks = -(-n // _C)
    last_base = n - _C
    trips = -(-num_chunks // _NW)
    span = trips * _C
    idx_base_max = n - span
    ncg = d // _L

    mesh = plsc.VectorSubcoreMesh(
        core_axis_name="c", subcore_axis_name="s",
        num_cores=_NC, num_subcores=_NS,
    )

    @functools.partial(
        pl.kernel,
        out_type=jax.ShapeDtypeStruct((n * d,), jnp.float32),
        mesh=mesh,
        scratch_types=[
            pltpu.VMEM((span + _L,), jnp.int32),
            pltpu.VMEM((b * d,), jnp.float32),
            pltpu.VMEM((_C * d,), jnp.float32),
            pltpu.SemaphoreType.DMA,
        ],
    )
    def bcast(idx_hbm, glob_hbm, out_hbm, idx_v, glob_v, rows_v, sem_w):
        wid = lax.axis_index("s") * _NC + lax.axis_index("c")
        c0 = (wid * num_chunks) // _NW
        idx_base = jnp.minimum(c0 * _C, idx_base_max)

        cp_g = pltpu.async_copy(glob_hbm, glob_v, sem_w)
        cp_i = pltpu.async_copy(idx_hbm.at[pl.ds(idx_base, span)],
                                idx_v.at[pl.ds(0, span)], sem_w)
        cp_g.wait()
        cp_i.wait()

        def fill_uniform(bid):
            row = [glob_v[pl.ds(bid * d + k * _L, _L)] for k in range(ncg)]

            def store_row(r, carry):
                for k in range(ncg):
                    rows_v[pl.ds(r * d + k * _L, _L)] = row[k]
                return carry
            lax.fori_loop(0, _C, store_row, 0)

        def fill_mixed(voff):
            def store_row(r, carry):
                bid = idx_v[pl.ds(voff + r, _L)][0]
                for k in range(ncg):
                    rows_v[pl.ds(r * d + k * _L, _L)] = (
                        glob_v[pl.ds(bid * d + k * _L, _L)])
                return carry
            lax.fori_loop(0, _C, store_row, 0)

        def drain(k):
            def wait_one(_, carry):
                pltpu.make_async_copy(
                    rows_v, out_hbm.at[pl.ds(0, _C * d)], sem_w).wait()
                return carry
            lax.fori_loop(0, k, wait_one, 0)

        def body(j, carry):
            buf_id, n_out = carry
            cc = jnp.minimum(c0 + j, num_chunks - 1)
            base = jnp.minimum(cc * _C, last_base)
            voff = base - idx_base

            lo = idx_v[pl.ds(voff, _L)][0]
            hi = idx_v[pl.ds(voff + _C - _L, _L)][_L - 1]
            uniform = lo == hi
            stale = jnp.logical_not(jnp.logical_and(uniform, lo == buf_id))

            @pl.when(stale)
            def _():
                drain(n_out)

            @pl.when(jnp.logical_and(stale, uniform))
            def _():
                fill_uniform(lo)

            @pl.when(jnp.logical_not(uniform))
            def _():
                fill_mixed(voff)

            pltpu.async_copy(rows_v, out_hbm.at[pl.ds(base * d, _C * d)],
                             sem_w)
            new_id = jnp.where(uniform, lo, jnp.int32(-2))
            new_out = jnp.where(stale, jnp.int32(1), n_out + 1)
            return new_id, new_out

        _, n_left = lax.fori_loop(
            0, trips, body, (jnp.int32(-2), jnp.int32(0)))
        drain(n_left)

    out = bcast(idx, glob_flat)
    return out.reshape(n, d)


# speedup vs baseline: 1.6156x; 1.0843x over previous
import functools

import jax
import jax.numpy as jnp
from jax import lax
from jax.experimental import pallas as pl
from jax.experimental.pallas import tpu as pltpu
from jax.experimental.pallas import tpu_sc as plsc

_NC = 2
_NS = 16
_NW = _NC * _NS
_L = 16

_C = 64


def kernel(feat, batch_ids, glob_feat):
    n = feat.shape[0]
    b, d = glob_feat.shape
    idx = batch_ids.astype(jnp.int32)
    glob_flat = glob_feat.reshape(-1)

    num_chunks = -(-n // _C)
    last_base = n - _C
    trips = -(-num_chunks // _NW)
    span = trips * _C
    idx_base_max = n - span
    ncg = d // _L

    mesh = plsc.VectorSubcoreMesh(
        core_axis_name="c", subcore_axis_name="s",
        num_cores=_NC, num_subcores=_NS,
    )

    @functools.partial(
        pl.kernel,
        out_type=jax.ShapeDtypeStruct((n * d,), jnp.float32),
        mesh=mesh,
        scratch_types=[
            pltpu.VMEM((span + _L,), jnp.int32),
            pltpu.VMEM((b * d,), jnp.float32),
            pltpu.VMEM((_C * d,), jnp.float32),
            pltpu.SemaphoreType.DMA,
        ],
    )
    def bcast(idx_hbm, glob_hbm, out_hbm, idx_v, glob_v, rows_v, sem_w):
        wid = lax.axis_index("s") * _NC + lax.axis_index("c")
        c0 = (wid * num_chunks) // _NW
        idx_base = jnp.minimum(c0 * _C, idx_base_max)

        cp_g = pltpu.async_copy(glob_hbm, glob_v, sem_w)
        cp_i = pltpu.async_copy(idx_hbm.at[pl.ds(idx_base, span)],
                                idx_v.at[pl.ds(0, span)], sem_w)
        cp_g.wait()
        cp_i.wait()

        def fill_uniform(bid):
            row = [glob_v[pl.ds(bid * d + k * _L, _L)] for k in range(ncg)]

            def store_row(r, carry):
                for k in range(ncg):
                    rows_v[pl.ds(r * d + k * _L, _L)] = row[k]
                return carry
            lax.fori_loop(0, _C, store_row, 0)

        def fill_mixed(voff):
            def store_row(r, carry):
                bid = idx_v[pl.ds(voff + r, _L)][0]
                for k in range(ncg):
                    rows_v[pl.ds(r * d + k * _L, _L)] = (
                        glob_v[pl.ds(bid * d + k * _L, _L)])
                return carry
            lax.fori_loop(0, _C, store_row, 0)

        def drain(k):
            def wait_one(_, carry):
                pltpu.make_async_copy(
                    rows_v, out_hbm.at[pl.ds(0, _C * d)], sem_w).wait()
                return carry
            lax.fori_loop(0, k, wait_one, 0)

        def body(j, carry):
            buf_id, n_out = carry
            cc = jnp.minimum(c0 + j, num_chunks - 1)
            base = jnp.minimum(cc * _C, last_base)
            voff = base - idx_base

            lo = idx_v[pl.ds(voff, _L)][0]
            hi = idx_v[pl.ds(voff + _C - _L, _L)][_L - 1]
            uniform = lo == hi
            stale = jnp.logical_not(jnp.logical_and(uniform, lo == buf_id))

            @pl.when(stale)
            def _():
                drain(n_out)

            @pl.when(jnp.logical_and(stale, uniform))
            def _():
                fill_uniform(lo)

            @pl.when(jnp.logical_not(uniform))
            def _():
                fill_mixed(voff)

            pltpu.async_copy(rows_v, out_hbm.at[pl.ds(base * d, _C * d)],
                             sem_w)
            new_id = jnp.where(uniform, lo, jnp.int32(-2))
            new_out = jnp.where(stale, jnp.int32(1), n_out + 1)
            return new_id, new_out

        _, n_left = lax.fori_loop(
            0, trips, body, (jnp.int32(-2), jnp.int32(0)))
        drain(n_left)

    out = bcast(idx, glob_flat)
    return out.reshape(n, d)


# speedup vs baseline: 1.6669x; 1.0317x over previous
import functools

import jax
import jax.numpy as jnp
from jax import lax
from jax.experimental import pallas as pl
from jax.experimental.pallas import tpu as pltpu
from jax.experimental.pallas import tpu_sc as plsc

_NC = 2
_NS = 16
_NW = _NC * _NS
_L = 16

_C = 32


def kernel(feat, batch_ids, glob_feat):
    n = feat.shape[0]
    b, d = glob_feat.shape
    idx = batch_ids.astype(jnp.int32)
    glob_flat = glob_feat.reshape(-1)

    num_chunks = -(-n // _C)
    last_base = n - _C
    trips = -(-num_chunks // _NW)
    span = trips * _C
    idx_base_max = n - span
    ncg = d // _L

    mesh = plsc.VectorSubcoreMesh(
        core_axis_name="c", subcore_axis_name="s",
        num_cores=_NC, num_subcores=_NS,
    )

    @functools.partial(
        pl.kernel,
        out_type=jax.ShapeDtypeStruct((n * d,), jnp.float32),
        mesh=mesh,
        scratch_types=[
            pltpu.VMEM((span + _L,), jnp.int32),
            pltpu.VMEM((b * d,), jnp.float32),
            pltpu.VMEM((_C * d,), jnp.float32),
            pltpu.SemaphoreType.DMA,
        ],
    )
    def bcast(idx_hbm, glob_hbm, out_hbm, idx_v, glob_v, rows_v, sem_w):
        wid = lax.axis_index("s") * _NC + lax.axis_index("c")
        c0 = (wid * num_chunks) // _NW
        idx_base = jnp.minimum(c0 * _C, idx_base_max)

        cp_g = pltpu.async_copy(glob_hbm, glob_v, sem_w)
        cp_i = pltpu.async_copy(idx_hbm.at[pl.ds(idx_base, span)],
                                idx_v.at[pl.ds(0, span)], sem_w)
        cp_g.wait()
        cp_i.wait()

        def fill_uniform(bid):
            row = [glob_v[pl.ds(bid * d + k * _L, _L)] for k in range(ncg)]

            def store_row(r, carry):
                for k in range(ncg):
                    rows_v[pl.ds(r * d + k * _L, _L)] = row[k]
                return carry
            lax.fori_loop(0, _C, store_row, 0)

        def fill_mixed(voff):
            def store_row(r, carry):
                bid = idx_v[pl.ds(voff + r, _L)][0]
                for k in range(ncg):
                    rows_v[pl.ds(r * d + k * _L, _L)] = (
                        glob_v[pl.ds(bid * d + k * _L, _L)])
                return carry
            lax.fori_loop(0, _C, store_row, 0)

        def drain(k):
            def wait_one(_, carry):
                pltpu.make_async_copy(
                    rows_v, out_hbm.at[pl.ds(0, _C * d)], sem_w).wait()
                return carry
            lax.fori_loop(0, k, wait_one, 0)

        def body(j, carry):
            buf_id, n_out = carry
            cc = jnp.minimum(c0 + j, num_chunks - 1)
            base = jnp.minimum(cc * _C, last_base)
            voff = base - idx_base

            lo = idx_v[pl.ds(voff, _L)][0]
            hi = idx_v[pl.ds(voff + _C - _L, _L)][_L - 1]
            uniform = lo == hi
            stale = jnp.logical_not(jnp.logical_and(uniform, lo == buf_id))

            @pl.when(stale)
            def _():
                drain(n_out)

            @pl.when(jnp.logical_and(stale, uniform))
            def _():
                fill_uniform(lo)

            @pl.when(jnp.logical_not(uniform))
            def _():
                fill_mixed(voff)

            pltpu.async_copy(rows_v, out_hbm.at[pl.ds(base * d, _C * d)],
                             sem_w)
            new_id = jnp.where(uniform, lo, jnp.int32(-2))
            new_out = jnp.where(stale, jnp.int32(1), n_out + 1)
            return new_id, new_out

        _, n_left = lax.fori_loop(
            0, trips, body, (jnp.int32(-2), jnp.int32(0)))
        drain(n_left)

    out = bcast(idx, glob_flat)
    return out.reshape(n, d)


# speedup vs baseline: 1.6816x; 1.0088x over previous
import functools

import jax
import jax.numpy as jnp
from jax import lax
from jax.experimental import pallas as pl
from jax.experimental.pallas import tpu as pltpu
from jax.experimental.pallas import tpu_sc as plsc

_NC = 2
_NS = 16
_NW = _NC * _NS
_L = 16

_C = 16


def kernel(feat, batch_ids, glob_feat):
    n = feat.shape[0]
    b, d = glob_feat.shape
    idx = batch_ids.astype(jnp.int32)
    glob_flat = glob_feat.reshape(-1)

    num_chunks = -(-n // _C)
    last_base = n - _C
    trips = -(-num_chunks // _NW)
    span = trips * _C
    idx_base_max = n - span
    ncg = d // _L

    mesh = plsc.VectorSubcoreMesh(
        core_axis_name="c", subcore_axis_name="s",
        num_cores=_NC, num_subcores=_NS,
    )

    @functools.partial(
        pl.kernel,
        out_type=jax.ShapeDtypeStruct((n * d,), jnp.float32),
        mesh=mesh,
        scratch_types=[
            pltpu.VMEM((span + _L,), jnp.int32),
            pltpu.VMEM((b * d,), jnp.float32),
            pltpu.VMEM((_C * d,), jnp.float32),
            pltpu.SemaphoreType.DMA,
        ],
    )
    def bcast(idx_hbm, glob_hbm, out_hbm, idx_v, glob_v, rows_v, sem_w):
        wid = lax.axis_index("s") * _NC + lax.axis_index("c")
        c0 = (wid * num_chunks) // _NW
        idx_base = jnp.minimum(c0 * _C, idx_base_max)

        cp_g = pltpu.async_copy(glob_hbm, glob_v, sem_w)
        cp_i = pltpu.async_copy(idx_hbm.at[pl.ds(idx_base, span)],
                                idx_v.at[pl.ds(0, span)], sem_w)
        cp_g.wait()
        cp_i.wait()

        def fill_uniform(bid):
            row = [glob_v[pl.ds(bid * d + k * _L, _L)] for k in range(ncg)]

            def store_row(r, carry):
                for k in range(ncg):
                    rows_v[pl.ds(r * d + k * _L, _L)] = row[k]
                return carry
            lax.fori_loop(0, _C, store_row, 0)

        def fill_mixed(voff):
            def store_row(r, carry):
                bid = idx_v[pl.ds(voff + r, _L)][0]
                for k in range(ncg):
                    rows_v[pl.ds(r * d + k * _L, _L)] = (
                        glob_v[pl.ds(bid * d + k * _L, _L)])
                return carry
            lax.fori_loop(0, _C, store_row, 0)

        def drain(k):
            def wait_one(_, carry):
                pltpu.make_async_copy(
                    rows_v, out_hbm.at[pl.ds(0, _C * d)], sem_w).wait()
                return carry
            lax.fori_loop(0, k, wait_one, 0)

        def body(j, carry):
            buf_id, n_out = carry
            cc = jnp.minimum(c0 + j, num_chunks - 1)
            base = jnp.minimum(cc * _C, last_base)
            voff = base - idx_base

            lo = idx_v[pl.ds(voff, _L)][0]
            hi = idx_v[pl.ds(voff + _C - _L, _L)][_L - 1]
            uniform = lo == hi
            stale = jnp.logical_not(jnp.logical_and(uniform, lo == buf_id))

            @pl.when(stale)
            def _():
                drain(n_out)

            @pl.when(jnp.logical_and(stale, uniform))
            def _():
                fill_uniform(lo)

            @pl.when(jnp.logical_not(uniform))
            def _():
                fill_mixed(voff)

            pltpu.async_copy(rows_v, out_hbm.at[pl.ds(base * d, _C * d)],
                             sem_w)
            new_id = jnp.where(uniform, lo, jnp.int32(-2))
            new_out = jnp.where(stale, jnp.int32(1), n_out + 1)
            return new_id, new_out

        _, n_left = lax.fori_loop(
            0, trips, body, (jnp.int32(-2), jnp.int32(0)))
        drain(n_left)

    out = bcast(idx, glob_flat)
    return out.reshape(n, d)
